# baseline (speedup 1.0000x reference)
import functools

import jax
import jax.numpy as jnp
from jax import lax
from jax.experimental import pallas as pl
from jax.experimental.pallas import tpu as pltpu
from jax.experimental.pallas import tpu_sc as plsc

_B = 128
_LANES = 16
_NS = 16
_NC = 2


def _make_sc_agg(n, e_pad, d, r_rel, n_pad, rpt):
    trash = n
    n_rounds = r_rel // _NC

    mesh = plsc.VectorSubcoreMesh(core_axis_name="c", subcore_axis_name="s")

    @functools.partial(
        pl.kernel,
        out_type=(
            jax.ShapeDtypeStruct((r_rel, n_pad, d), jnp.float32),
            jax.ShapeDtypeStruct((r_rel, n_pad, _LANES), jnp.float32),
        ),
        mesh=mesh,
        compiler_params=pltpu.CompilerParams(use_tc_tiling_on_sc=False),
        scratch_types=[
            pltpu.VMEM((32,), jnp.int32),
            pltpu.VMEM((2, _B), jnp.int32),
            pltpu.VMEM((2, _B), jnp.int32),
            pltpu.VMEM((_B, d), jnp.float32),
            pltpu.VMEM((_B, d), jnp.float32),
            pltpu.VMEM((_B, _LANES), jnp.float32),
            pltpu.VMEM_SHARED((n_pad, d), jnp.float32),
            pltpu.VMEM_SHARED((n_pad, _LANES), jnp.float32),
            pltpu.SemaphoreType.DMA,
            pltpu.SemaphoreType.DMA,
        ],
    )
    def sc_agg(h_hbm, ei_hbm, offs_hbm, agg_hbm, deg_hbm,
               offs_v, eb0, eb1, rows0, rows1, ones_v, agg_sp, deg_sp,
               sem0, sem1):
        c = lax.axis_index("c")
        s = lax.axis_index("s")
        lane = lax.iota(jnp.int32, _LANES)

        pltpu.sync_copy(offs_hbm, offs_v)
        zvec = jnp.zeros((_LANES,), jnp.float32)
        onev = jnp.ones((_LANES,), jnp.float32)

        def onerow(i, carry):
            ones_v[i, pl.ds(0, _LANES)] = onev
            return carry

        lax.fori_loop(0, _B, onerow, 0)

        for rnd in range(n_rounds):
            r = rnd * _NC + c

            def zrow(i, carry):
                for j in range(d // _LANES):
                    rows0[i, pl.ds(j * _LANES, _LANES)] = zvec
                return carry

            lax.fori_loop(0, _B, zrow, 0)
            row0 = s * rpt
            nfull = rpt // _B
            rem = rpt - nfull * _B
            for kz in range(nfull):
                pltpu.sync_copy(rows0, agg_sp.at[pl.ds(row0 + kz * _B, _B)])
                pltpu.sync_copy(rows0.at[:, pl.ds(0, _LANES)],
                                deg_sp.at[pl.ds(row0 + kz * _B, _B)])
            if rem:
                pltpu.sync_copy(rows0.at[pl.ds(0, rem)],
                                agg_sp.at[pl.ds(row0 + nfull * _B, rem)])
                pltpu.sync_copy(rows0.at[pl.ds(0, rem), pl.ds(0, _LANES)],
                                deg_sp.at[pl.ds(row0 + nfull * _B, rem)])
            plsc.subcore_barrier()

            ov = offs_v[pl.ds(r, _LANES)]
            off = ov[0]
            cnt = ov[1] - off
            per_tile = (cnt + _NS - 1) // _NS
            my_start = off + s * per_tile
            my_cnt = jnp.maximum(jnp.minimum(per_tile, cnt - s * per_tile), 0)
            astart = (my_start // 8) * 8
            shift = my_start - astart
            total = shift + my_cnt
            nb = (total + _B - 1) // _B
            nbp = jnp.maximum((nb + 1) // 2, 1)

            def stage(k, eb, rows, sem):
                base = astart + k * _B
                pltpu.sync_copy(ei_hbm.at[:, pl.ds(base, _B)], eb)
                pltpu.async_copy(h_hbm.at[eb.at[0]], rows, sem)
                for j in range(_B // _LANES):
                    pos = lane + (k * _B + j * _LANES)
                    ok = (pos >= shift) & (pos < total)
                    dv = eb[1, pl.ds(j * _LANES, _LANES)]
                    eb[1, pl.ds(j * _LANES, _LANES)] = jnp.where(ok, dv, trash)

            def finish(eb, rows, sem):
                pltpu.make_async_copy(h_hbm.at[eb.at[0]], rows, sem).wait()
                pltpu.sync_copy(rows, agg_sp.at[eb.at[1]], add=True)
                pltpu.sync_copy(ones_v, deg_sp.at[eb.at[1]], add=True)

            stage(0, eb0, rows0, sem0)

            def pair(i, carry):
                stage(2 * i + 1, eb1, rows1, sem1)
                finish(eb0, rows0, sem0)
                stage(2 * i + 2, eb0, rows0, sem0)
                finish(eb1, rows1, sem1)
                return carry

            lax.fori_loop(0, nbp, pair, 0)
            finish(eb0, rows0, sem0)
            plsc.subcore_barrier()

            pltpu.sync_copy(agg_sp.at[pl.ds(row0, rpt)],
                            agg_hbm.at[r, pl.ds(row0, rpt)])
            pltpu.sync_copy(deg_sp.at[pl.ds(row0, rpt)],
                            deg_hbm.at[r, pl.ds(row0, rpt)])
            plsc.subcore_barrier()

    return sc_agg


def _make_tc_dense(n, n_pad, d_in, d_out, r_rel, nb_basis, blk, act):
    grid = n // blk

    def body(h_ref, agg_ref, deg_ref, v_ref, c_ref, l_ref, b_ref, o_ref):
        acc = jnp.dot(h_ref[...], l_ref[...],
                      preferred_element_type=jnp.float32)
        acc = acc + b_ref[...]
        for r in range(r_rel):
            w = c_ref[r, 0] * v_ref[0]
            for bi in range(1, nb_basis):
                w = w + c_ref[r, bi] * v_ref[bi]
            deg = deg_ref[:, r:r + 1]
            a = agg_ref[r] / jnp.maximum(deg, 1.0)
            acc = acc + jnp.dot(a, w, preferred_element_type=jnp.float32)
        if act:
            acc = jnp.maximum(acc, 0.0)
        o_ref[...] = acc

    return pl.pallas_call(
        body,
        grid=(grid,),
        in_specs=[
            pl.BlockSpec((blk, d_in), lambda i: (i, 0)),
            pl.BlockSpec((r_rel, blk, d_in), lambda i: (0, i, 0)),
            pl.BlockSpec((blk, r_rel), lambda i: (i, 0)),
            pl.BlockSpec((nb_basis, d_in, d_out), lambda i: (0, 0, 0)),
            pl.BlockSpec(memory_space=pltpu.SMEM),
            pl.BlockSpec((d_in, d_out), lambda i: (0, 0)),
            pl.BlockSpec((1, d_out), lambda i: (0, 0)),
        ],
        out_specs=pl.BlockSpec((blk, d_out), lambda i: (i, 0)),
        out_shape=jax.ShapeDtypeStruct((n, d_out), jnp.float32),
    )


def kernel(x, edge_index, edge_type, V1, c1, L1, b1, V2, c2, L2, b2,
           V3, c3, L3, b3):
    n, d_in = x.shape
    e = edge_index.shape[1]
    r_rel, nb_basis = c1.shape
    d_out = L3.shape[1]
    rpt = ((n + 8 + _NS * 8 - 1) // (_NS * 8)) * 8
    n_pad = rpt * _NS
    e_pad = e + 4 * _B

    pad = jnp.stack([jnp.zeros((e_pad - e,), edge_index.dtype),
                     jnp.full((e_pad - e,), n, edge_index.dtype)])
    ei_pad = jnp.concatenate([edge_index, pad], axis=1)
    offs = jnp.sum(edge_type[None, :] < jnp.arange(r_rel + 1)[:, None],
                   axis=1, dtype=jnp.int32)
    offs16 = jnp.zeros((32,), jnp.int32).at[: r_rel + 1].set(offs)

    sc_agg = _make_sc_agg(n, e_pad, d_in, r_rel, n_pad, rpt)
    blk = 1000 if n % 1000 == 0 else 8

    h = x
    layers = [(V1, c1, L1, b1, True),
              (V2, c2, L2, b2, True),
              (V3, c3, L3, b3, False)]
    for V, c, L, b, act in layers:
        agg, deg = sc_agg(h, ei_pad, offs16)
        tc = _make_tc_dense(n, n_pad, d_in, d_out, r_rel, nb_basis, blk, act)
        h = tc(h, agg, deg[:, :, 0].T, V, c, L, b.reshape(1, -1))
    return h


# speedup vs baseline: 1.3511x; 1.3511x over previous
import functools

import jax
import jax.numpy as jnp
from jax import lax
from jax.experimental import pallas as pl
from jax.experimental.pallas import tpu as pltpu
from jax.experimental.pallas import tpu_sc as plsc

_B = 128
_LANES = 16
_NS = 16
_NC = 2


def _make_sc_agg(n, e_pad, d, r_rel, n_pad, rpt, with_deg):
    trash = n
    n_rounds = r_rel // _NC
    agg_bytes = _B * d * 4
    deg_bytes = _B * _LANES * 4

    mesh = plsc.VectorSubcoreMesh(core_axis_name="c", subcore_axis_name="s")

    out_type = [jax.ShapeDtypeStruct((r_rel, n_pad, d), jnp.float32)]
    scratch = [
        pltpu.VMEM((32,), jnp.int32),
        pltpu.VMEM((2, _B), jnp.int32),
        pltpu.VMEM((2, _B), jnp.int32),
        pltpu.VMEM((_B, d), jnp.float32),
        pltpu.VMEM((_B, d), jnp.float32),
        pltpu.VMEM_SHARED((n_pad, d), jnp.float32),
        pltpu.SemaphoreType.DMA,
        pltpu.SemaphoreType.DMA,
        pltpu.SemaphoreType.DMA,
        pltpu.SemaphoreType.DMA,
    ]
    if with_deg:
        out_type.append(
            jax.ShapeDtypeStruct((r_rel, n_pad, _LANES), jnp.float32))
        scratch.append(pltpu.VMEM((_B, _LANES), jnp.float32))
        scratch.append(pltpu.VMEM_SHARED((n_pad, _LANES), jnp.float32))

    @functools.partial(
        pl.kernel,
        out_type=tuple(out_type),
        mesh=mesh,
        compiler_params=pltpu.CompilerParams(use_tc_tiling_on_sc=False),
        scratch_types=scratch,
    )
    def sc_agg(h_hbm, ei_hbm, offs_hbm, agg_hbm, *rest):
        if with_deg:
            (deg_hbm, offs_v, eb0, eb1, rows0, rows1, agg_sp,
             g0, g1, s0, s1, ones_v, deg_sp) = rest
        else:
            (offs_v, eb0, eb1, rows0, rows1, agg_sp, g0, g1, s0, s1) = rest
        c = lax.axis_index("c")
        s = lax.axis_index("s")
        lane = lax.iota(jnp.int32, _LANES)

        pltpu.sync_copy(offs_hbm, offs_v)
        zvec = jnp.zeros((_LANES,), jnp.float32)

        if with_deg:
            onev = jnp.ones((_LANES,), jnp.float32)

            def onerow(i, carry):
                ones_v[i, pl.ds(0, _LANES)] = onev
                return carry

            lax.fori_loop(0, _B, onerow, 0)

        buf_bytes = agg_bytes + (deg_bytes if with_deg else 0)

        def stage(k, eb, rows, gsem, ssem, first):
            if not first:
                pltpu.make_async_copy(rows, agg_sp.at[eb.at[1]], ssem).wait()
                if with_deg:
                    pltpu.make_async_copy(
                        ones_v, deg_sp.at[eb.at[1]], ssem).wait()
            base = astart + k * _B
            pltpu.sync_copy(ei_hbm.at[:, pl.ds(base, _B)], eb)
            pltpu.async_copy(h_hbm.at[eb.at[0]], rows, gsem)
            for j in range(_B // _LANES):
                pos = lane + (k * _B + j * _LANES)
                ok = (pos >= shift) & (pos < total)
                dv = eb[1, pl.ds(j * _LANES, _LANES)]
                eb[1, pl.ds(j * _LANES, _LANES)] = jnp.where(ok, dv, trash)

        def finish(eb, rows, gsem, ssem):
            pltpu.make_async_copy(h_hbm.at[eb.at[0]], rows, gsem).wait()
            pltpu.async_copy(rows, agg_sp.at[eb.at[1]], ssem, add=True)
            if with_deg:
                pltpu.async_copy(ones_v, deg_sp.at[eb.at[1]], ssem, add=True)

        for rnd in range(n_rounds):
            r = rnd * _NC + c

            def zrow(i, carry):
                for j in range(d // _LANES):
                    rows0[i, pl.ds(j * _LANES, _LANES)] = zvec
                return carry

            lax.fori_loop(0, _B, zrow, 0)
            row0 = s * rpt
            nfull = rpt // _B
            rem = rpt - nfull * _B
            for kz in range(nfull):
                pltpu.sync_copy(rows0, agg_sp.at[pl.ds(row0 + kz * _B, _B)])
                if with_deg:
                    pltpu.sync_copy(rows0.at[:, pl.ds(0, _LANES)],
                                    deg_sp.at[pl.ds(row0 + kz * _B, _B)])
            if rem:
                pltpu.sync_copy(rows0.at[pl.ds(0, rem)],
                                agg_sp.at[pl.ds(row0 + nfull * _B, rem)])
                if with_deg:
                    pltpu.sync_copy(rows0.at[pl.ds(0, rem), pl.ds(0, _LANES)],
                                    deg_sp.at[pl.ds(row0 + nfull * _B, rem)])
            plsc.subcore_barrier()

            ov = offs_v[pl.ds(r, _LANES)]
            off = ov[0]
            cnt = ov[1] - off
            per_tile = (cnt + _NS - 1) // _NS
            my_start = off + s * per_tile
            my_cnt = jnp.maximum(jnp.minimum(per_tile, cnt - s * per_tile), 0)
            astart = (my_start // 8) * 8
            shift = my_start - astart
            total = shift + my_cnt
            nb = (total + _B - 1) // _B
            nbp = jnp.maximum((nb + 1) // 2, 1)

            stage(0, eb0, rows0, g0, s0, True)
            stage(1, eb1, rows1, g1, s1, True)

            def pair(i, carry):
                finish(eb0, rows0, g0, s0)
                stage(2 * i + 2, eb0, rows0, g0, s0, False)
                finish(eb1, rows1, g1, s1)
                stage(2 * i + 3, eb1, rows1, g1, s1, False)
                return carry

            lax.fori_loop(0, nbp, pair, 0)
            finish(eb0, rows0, g0, s0)
            finish(eb1, rows1, g1, s1)
            pltpu.make_async_copy(rows0, agg_sp.at[eb0.at[1]], s0).wait()
            pltpu.make_async_copy(rows1, agg_sp.at[eb1.at[1]], s1).wait()
            if with_deg:
                pltpu.make_async_copy(ones_v, deg_sp.at[eb0.at[1]], s0).wait()
                pltpu.make_async_copy(ones_v, deg_sp.at[eb1.at[1]], s1).wait()
            plsc.subcore_barrier()

            pltpu.sync_copy(agg_sp.at[pl.ds(row0, rpt)],
                            agg_hbm.at[r, pl.ds(row0, rpt)])
            if with_deg:
                pltpu.sync_copy(deg_sp.at[pl.ds(row0, rpt)],
                                deg_hbm.at[r, pl.ds(row0, rpt)])
            plsc.subcore_barrier()

    return sc_agg


def _make_tc_dense(n, n_pad, d_in, d_out, r_rel, nb_basis, blk, act):
    grid = n // blk

    def body(h_ref, agg_ref, deg_ref, v_ref, c_ref, l_ref, b_ref, o_ref):
        acc = jnp.dot(h_ref[...], l_ref[...],
                      preferred_element_type=jnp.float32)
        acc = acc + b_ref[...]
        for r in range(r_rel):
            w = c_ref[r, 0] * v_ref[0]
            for bi in range(1, nb_basis):
                w = w + c_ref[r, bi] * v_ref[bi]
            deg = deg_ref[r][:, 0:1]
            a = agg_ref[r] / jnp.maximum(deg, 1.0)
            acc = acc + jnp.dot(a, w, preferred_element_type=jnp.float32)
        if act:
            acc = jnp.maximum(acc, 0.0)
        o_ref[...] = acc

    return pl.pallas_call(
        body,
        grid=(grid,),
        in_specs=[
            pl.BlockSpec((blk, d_in), lambda i: (i, 0)),
            pl.BlockSpec((r_rel, blk, d_in), lambda i: (0, i, 0)),
            pl.BlockSpec((r_rel, blk, _LANES), lambda i: (0, i, 0)),
            pl.BlockSpec((nb_basis, d_in, d_out), lambda i: (0, 0, 0)),
            pl.BlockSpec(memory_space=pltpu.SMEM),
            pl.BlockSpec((d_in, d_out), lambda i: (0, 0)),
            pl.BlockSpec((1, d_out), lambda i: (0, 0)),
        ],
        out_specs=pl.BlockSpec((blk, d_out), lambda i: (i, 0)),
        out_shape=jax.ShapeDtypeStruct((n, d_out), jnp.float32),
    )


def kernel(x, edge_index, edge_type, V1, c1, L1, b1, V2, c2, L2, b2,
           V3, c3, L3, b3):
    n, d_in = x.shape
    e = edge_index.shape[1]
    r_rel, nb_basis = c1.shape
    d_out = L3.shape[1]
    rpt = ((n + 8 + _NS * 8 - 1) // (_NS * 8)) * 8
    n_pad = rpt * _NS
    e_pad = e + 8 * _B

    pad = jnp.stack([jnp.zeros((e_pad - e,), edge_index.dtype),
                     jnp.full((e_pad - e,), n, edge_index.dtype)])
    ei_pad = jnp.concatenate([edge_index, pad], axis=1)
    offs = jnp.sum(edge_type[None, :] < jnp.arange(r_rel + 1)[:, None],
                   axis=1, dtype=jnp.int32)
    offs16 = jnp.zeros((32,), jnp.int32).at[: r_rel + 1].set(offs)

    sc_agg_d = _make_sc_agg(n, e_pad, d_in, r_rel, n_pad, rpt, True)
    sc_agg = _make_sc_agg(n, e_pad, d_in, r_rel, n_pad, rpt, False)
    blk = 1000 if n % 1000 == 0 else 8

    layers = [(V1, c1, L1, b1, True),
              (V2, c2, L2, b2, True),
              (V3, c3, L3, b3, False)]
    h, deg = x, None
    for li, (V, c, L, b, act) in enumerate(layers):
        if li == 0:
            agg, deg = sc_agg_d(h, ei_pad, offs16)
        else:
            (agg,) = sc_agg(h, ei_pad, offs16)
        tc = _make_tc_dense(n, n_pad, d_in, d_out, r_rel, nb_basis, blk, act)
        h = tc(h, agg, deg, V, c, L, b.reshape(1, -1))
    return h


# speedup vs baseline: 1.4744x; 1.0913x over previous
import functools

import jax
import jax.numpy as jnp
from jax import lax
from jax.experimental import pallas as pl
from jax.experimental.pallas import tpu as pltpu
from jax.experimental.pallas import tpu_sc as plsc

_B = 128
_LANES = 16
_NS = 16
_NC = 2


def _make_sc_agg(n, e, d, r_rel, n_pad, rpt, with_deg):
    trash = n
    n_rounds = r_rel // _NC

    mesh = plsc.VectorSubcoreMesh(core_axis_name="c", subcore_axis_name="s")

    out_type = [jax.ShapeDtypeStruct((r_rel, n_pad, d), jnp.float32)]
    scratch = [
        pltpu.VMEM((32,), jnp.int32),
        pltpu.VMEM((2, _B), jnp.int32),
        pltpu.VMEM((2, _B), jnp.int32),
        pltpu.VMEM((2, _B), jnp.int32),
        pltpu.VMEM((2, _B), jnp.int32),
        pltpu.VMEM((_B, d), jnp.float32),
        pltpu.VMEM((_B, d), jnp.float32),
        pltpu.VMEM_SHARED((n_pad, d), jnp.float32),
        pltpu.SemaphoreType.DMA,
        pltpu.SemaphoreType.DMA,
        pltpu.SemaphoreType.DMA,
        pltpu.SemaphoreType.DMA,
        pltpu.SemaphoreType.DMA,
        pltpu.SemaphoreType.DMA,
        pltpu.SemaphoreType.DMA,
        pltpu.SemaphoreType.DMA,
    ]
    if with_deg:
        out_type.append(
            jax.ShapeDtypeStruct((r_rel, n_pad, _LANES), jnp.float32))
        scratch.append(pltpu.VMEM((_B, _LANES), jnp.float32))
        scratch.append(pltpu.VMEM_SHARED((n_pad, _LANES), jnp.float32))

    @functools.partial(
        pl.kernel,
        out_type=tuple(out_type),
        mesh=mesh,
        compiler_params=pltpu.CompilerParams(use_tc_tiling_on_sc=False),
        scratch_types=scratch,
    )
    def sc_agg(h_hbm, ei_hbm, offs_hbm, agg_hbm, *rest):
        if with_deg:
            (deg_hbm, offs_v, eb0, eb1, eb2, eb3, rows0, rows1, agg_sp,
             g0, g1, s0, s1, i0, i1, i2, i3, ones_v, deg_sp) = rest
        else:
            (offs_v, eb0, eb1, eb2, eb3, rows0, rows1, agg_sp,
             g0, g1, s0, s1, i0, i1, i2, i3) = rest
        c = lax.axis_index("c")
        s = lax.axis_index("s")
        lane = lax.iota(jnp.int32, _LANES)

        pltpu.sync_copy(offs_hbm, offs_v)
        zvec = jnp.zeros((_LANES,), jnp.float32)

        if with_deg:
            onev = jnp.ones((_LANES,), jnp.float32)

            def onerow(i, carry):
                ones_v[i, pl.ds(0, _LANES)] = onev
                return carry

            lax.fori_loop(0, _B, onerow, 0)

        for rnd in range(n_rounds):
            r = rnd * _NC + c

            def zrow(i, carry):
                for j in range(d // _LANES):
                    rows0[i, pl.ds(j * _LANES, _LANES)] = zvec
                return carry

            lax.fori_loop(0, _B, zrow, 0)
            row0 = s * rpt
            nfull = rpt // _B
            rem = rpt - nfull * _B
            for kz in range(nfull):
                pltpu.sync_copy(rows0, agg_sp.at[pl.ds(row0 + kz * _B, _B)])
                if with_deg:
                    pltpu.sync_copy(rows0.at[:, pl.ds(0, _LANES)],
                                    deg_sp.at[pl.ds(row0 + kz * _B, _B)])
            if rem:
                pltpu.sync_copy(rows0.at[pl.ds(0, rem)],
                                agg_sp.at[pl.ds(row0 + nfull * _B, rem)])
                if with_deg:
                    pltpu.sync_copy(rows0.at[pl.ds(0, rem), pl.ds(0, _LANES)],
                                    deg_sp.at[pl.ds(row0 + nfull * _B, rem)])
            plsc.subcore_barrier()

            ov = offs_v[pl.ds(r, _LANES)]
            off = ov[0]
            cnt = ov[1] - off
            per_tile = (cnt + _NS - 1) // _NS
            my_start = off + s * per_tile
            my_cnt = jnp.maximum(jnp.minimum(per_tile, cnt - s * per_tile), 0)
            astart = (my_start // 8) * 8
            shift = my_start - astart
            total = shift + my_cnt
            nb = (total + _B - 1) // _B
            nq = jnp.maximum((nb + 3) // 4, 1)

            def fire_idx(k, eb, isem):
                pltpu.async_copy(
                    ei_hbm.at[:, pl.ds(astart + k * _B, _B)], eb, isem)

            def wait_idx(k, eb, isem):
                pltpu.make_async_copy(
                    ei_hbm.at[:, pl.ds(astart + k * _B, _B)], eb, isem).wait()

            def mask(k, eb):
                for j in range(_B // _LANES):
                    pos = lane + (k * _B + j * _LANES)
                    ok = (pos >= shift) & (pos < total)
                    dv = eb[1, pl.ds(j * _LANES, _LANES)]
                    eb[1, pl.ds(j * _LANES, _LANES)] = jnp.where(ok, dv, trash)

            def step(k, eb_k, eb_n, rows, gsem, ssem, isem_k):
                pltpu.make_async_copy(h_hbm.at[eb_k.at[0]], rows, gsem).wait()
                pltpu.async_copy(rows, agg_sp.at[eb_k.at[1]], ssem, add=True)
                if with_deg:
                    pltpu.async_copy(
                        ones_v, deg_sp.at[eb_k.at[1]], ssem, add=True)
                pltpu.make_async_copy(rows, agg_sp.at[eb_k.at[1]], ssem).wait()
                if with_deg:
                    pltpu.make_async_copy(
                        ones_v, deg_sp.at[eb_k.at[1]], ssem).wait()
                pltpu.async_copy(h_hbm.at[eb_n.at[0]], rows, gsem)
                fire_idx(k + 4, eb_k, isem_k)
                mask(k + 2, eb_n)

            fire_idx(0, eb0, i0)
            fire_idx(1, eb1, i1)
            wait_idx(0, eb0, i0)
            pltpu.async_copy(h_hbm.at[eb0.at[0]], rows0, g0)
            fire_idx(2, eb2, i2)
            mask(0, eb0)
            wait_idx(1, eb1, i1)
            pltpu.async_copy(h_hbm.at[eb1.at[0]], rows1, g1)
            fire_idx(3, eb3, i3)
            mask(1, eb1)

            def quad(i, carry):
                k = 4 * i
                wait_idx(k + 2, eb2, i2)
                step(k, eb0, eb2, rows0, g0, s0, i0)
                wait_idx(k + 3, eb3, i3)
                step(k + 1, eb1, eb3, rows1, g1, s1, i1)
                wait_idx(k + 4, eb0, i0)
                step(k + 2, eb2, eb0, rows0, g0, s0, i2)
                wait_idx(k + 5, eb1, i1)
                step(k + 3, eb3, eb1, rows1, g1, s1, i3)
                return carry

            lax.fori_loop(0, nq, quad, 0)

            pltpu.make_async_copy(h_hbm.at[eb0.at[0]], rows0, g0).wait()
            pltpu.async_copy(rows0, agg_sp.at[eb0.at[1]], s0, add=True)
            pltpu.make_async_copy(h_hbm.at[eb1.at[0]], rows1, g1).wait()
            pltpu.async_copy(rows1, agg_sp.at[eb1.at[1]], s1, add=True)
            pltpu.make_async_copy(rows0, agg_sp.at[eb0.at[1]], s0).wait()
            pltpu.make_async_copy(rows1, agg_sp.at[eb1.at[1]], s1).wait()
            if with_deg:
                pltpu.async_copy(ones_v, deg_sp.at[eb0.at[1]], s0, add=True)
                pltpu.async_copy(ones_v, deg_sp.at[eb1.at[1]], s1, add=True)
                pltpu.make_async_copy(ones_v, deg_sp.at[eb0.at[1]], s0).wait()
                pltpu.make_async_copy(ones_v, deg_sp.at[eb1.at[1]], s1).wait()
            wait_idx(4 * nq + 2, eb2, i2)
            wait_idx(4 * nq + 3, eb3, i3)
            plsc.subcore_barrier()

            pltpu.sync_copy(agg_sp.at[pl.ds(row0, rpt)],
                            agg_hbm.at[r, pl.ds(row0, rpt)])
            if with_deg:
                pltpu.sync_copy(deg_sp.at[pl.ds(row0, rpt)],
                                deg_hbm.at[r, pl.ds(row0, rpt)])
            plsc.subcore_barrier()

    return sc_agg


def _make_tc_dense(n, n_pad, d_in, d_out, r_rel, nb_basis, blk, act):
    grid = n // blk

    def body(h_ref, agg_ref, deg_ref, v_ref, c_ref, l_ref, b_ref, o_ref):
        acc = jnp.dot(h_ref[...], l_ref[...],
                      preferred_element_type=jnp.float32)
        acc = acc + b_ref[...]
        for r in range(r_rel):
            w = c_ref[r, 0] * v_ref[0]
            for bi in range(1, nb_basis):
                w = w + c_ref[r, bi] * v_ref[bi]
            deg = deg_ref[r][:, 0:1]
            a = agg_ref[r] / jnp.maximum(deg, 1.0)
            acc = acc + jnp.dot(a, w, preferred_element_type=jnp.float32)
        if act:
            acc = jnp.maximum(acc, 0.0)
        o_ref[...] = acc

    return pl.pallas_call(
        body,
        grid=(grid,),
        in_specs=[
            pl.BlockSpec((blk, d_in), lambda i: (i, 0)),
            pl.BlockSpec((r_rel, blk, d_in), lambda i: (0, i, 0)),
            pl.BlockSpec((r_rel, blk, _LANES), lambda i: (0, i, 0)),
            pl.BlockSpec((nb_basis, d_in, d_out), lambda i: (0, 0, 0)),
            pl.BlockSpec(memory_space=pltpu.SMEM),
            pl.BlockSpec((d_in, d_out), lambda i: (0, 0)),
            pl.BlockSpec((1, d_out), lambda i: (0, 0)),
        ],
        out_specs=pl.BlockSpec((blk, d_out), lambda i: (i, 0)),
        out_shape=jax.ShapeDtypeStruct((n, d_out), jnp.float32),
    )


def kernel(x, edge_index, edge_type, V1, c1, L1, b1, V2, c2, L2, b2,
           V3, c3, L3, b3):
    n, d_in = x.shape
    e = edge_index.shape[1]
    r_rel, nb_basis = c1.shape
    d_out = L3.shape[1]
    rpt = ((n + 8 + _NS * 8 - 1) // (_NS * 8)) * 8
    n_pad = rpt * _NS
    e_pad = e + 4096

    pad = jnp.stack([jnp.zeros((e_pad - e,), edge_index.dtype),
                     jnp.full((e_pad - e,), n, edge_index.dtype)])
    ei_pad = jnp.concatenate([edge_index, pad], axis=1)
    offs = jnp.sum(edge_type[None, :] < jnp.arange(r_rel + 1)[:, None],
                   axis=1, dtype=jnp.int32)
    offs16 = jnp.zeros((32,), jnp.int32).at[: r_rel + 1].set(offs)

    sc_agg_d = _make_sc_agg(n, e, d_in, r_rel, n_pad, rpt, True)
    sc_agg = _make_sc_agg(n, e, d_in, r_rel, n_pad, rpt, False)
    blk = 1000 if n % 1000 == 0 else 8

    layers = [(V1, c1, L1, b1, True),
              (V2, c2, L2, b2, True),
              (V3, c3, L3, b3, False)]
    h, deg = x, None
    for li, (V, c, L, b, act) in enumerate(layers):
        if li == 0:
            agg, deg = sc_agg_d(h, ei_pad, offs16)
        else:
            (agg,) = sc_agg(h, ei_pad, offs16)
        tc = _make_tc_dense(n, n_pad, d_in, d_out, r_rel, nb_basis, blk, act)
        h = tc(h, agg, deg, V, c, L, b.reshape(1, -1))
    return h


# speedup vs baseline: 1.4761x; 1.0011x over previous
import functools

import jax
import jax.numpy as jnp
from jax import lax
from jax.experimental import pallas as pl
from jax.experimental.pallas import tpu as pltpu
from jax.experimental.pallas import tpu_sc as plsc

_B = 128
_LANES = 16
_NS = 16
_NC = 2


def _make_sc_agg(n, e, d, r_rel, n_pad, rpt, with_deg):
    trash = n
    n_rounds = r_rel // _NC

    mesh = plsc.VectorSubcoreMesh(core_axis_name="c", subcore_axis_name="s")

    out_type = [jax.ShapeDtypeStruct((r_rel, n_pad, d), jnp.float32)]
    scratch = [
        pltpu.VMEM((32,), jnp.int32),
        pltpu.VMEM((2, _B), jnp.int32),
        pltpu.VMEM((2, _B), jnp.int32),
        pltpu.VMEM((2, _B), jnp.int32),
        pltpu.VMEM((2, _B), jnp.int32),
        pltpu.VMEM((_B, d), jnp.float32),
        pltpu.VMEM((_B, d), jnp.float32),
        pltpu.VMEM_SHARED((n_pad, d), jnp.float32),
        pltpu.SemaphoreType.DMA,
        pltpu.SemaphoreType.DMA,
        pltpu.SemaphoreType.DMA,
        pltpu.SemaphoreType.DMA,
        pltpu.SemaphoreType.DMA,
        pltpu.SemaphoreType.DMA,
        pltpu.SemaphoreType.DMA,
        pltpu.SemaphoreType.DMA,
    ]
    if with_deg:
        out_type.append(
            jax.ShapeDtypeStruct((r_rel, n_pad, _LANES), jnp.float32))
        scratch.append(pltpu.VMEM((_B, _LANES), jnp.float32))
        scratch.append(pltpu.VMEM_SHARED((n_pad, _LANES), jnp.float32))

    @functools.partial(
        pl.kernel,
        out_type=tuple(out_type),
        mesh=mesh,
        compiler_params=pltpu.CompilerParams(use_tc_tiling_on_sc=False),
        scratch_types=scratch,
    )
    def sc_agg(h_hbm, ei_hbm, offs_hbm, agg_hbm, *rest):
        if with_deg:
            (deg_hbm, offs_v, eb0, eb1, eb2, eb3, rows0, rows1, agg_sp,
             g0, g1, s0, s1, i0, i1, i2, i3, ones_v, deg_sp) = rest
        else:
            (offs_v, eb0, eb1, eb2, eb3, rows0, rows1, agg_sp,
             g0, g1, s0, s1, i0, i1, i2, i3) = rest
        c = lax.axis_index("c")
        s = lax.axis_index("s")
        lane = lax.iota(jnp.int32, _LANES)

        pltpu.sync_copy(offs_hbm, offs_v)
        zvec = jnp.zeros((_LANES,), jnp.float32)

        if with_deg:
            onev = jnp.ones((_LANES,), jnp.float32)

            def onerow(i, carry):
                ones_v[i, pl.ds(0, _LANES)] = onev
                return carry

            lax.fori_loop(0, _B, onerow, 0)

        for rnd in range(n_rounds):
            r = rnd * _NC + c

            def zrow(i, carry):
                for j in range(d // _LANES):
                    rows0[i, pl.ds(j * _LANES, _LANES)] = zvec
                return carry

            lax.fori_loop(0, _B, zrow, 0)
            row0 = s * rpt
            nfull = rpt // _B
            rem = rpt - nfull * _B
            for kz in range(nfull):
                pltpu.sync_copy(rows0, agg_sp.at[pl.ds(row0 + kz * _B, _B)])
                if with_deg:
                    pltpu.sync_copy(rows0.at[:, pl.ds(0, _LANES)],
                                    deg_sp.at[pl.ds(row0 + kz * _B, _B)])
            if rem:
                pltpu.sync_copy(rows0.at[pl.ds(0, rem)],
                                agg_sp.at[pl.ds(row0 + nfull * _B, rem)])
                if with_deg:
                    pltpu.sync_copy(rows0.at[pl.ds(0, rem), pl.ds(0, _LANES)],
                                    deg_sp.at[pl.ds(row0 + nfull * _B, rem)])
            plsc.subcore_barrier()

            ov = offs_v[pl.ds(r, _LANES)]
            off = ov[0]
            cnt = ov[1] - off
            per_tile = (cnt + _NS - 1) // _NS
            my_start = off + s * per_tile
            my_cnt = jnp.maximum(jnp.minimum(per_tile, cnt - s * per_tile), 0)
            astart = (my_start // 8) * 8
            shift = my_start - astart
            total = shift + my_cnt
            nb = (total + _B - 1) // _B
            nq = jnp.maximum((nb + 3) // 4, 1)

            def fire_idx(k, eb, isem):
                pltpu.async_copy(
                    ei_hbm.at[:, pl.ds(astart + k * _B, _B)], eb, isem)

            def wait_idx(k, eb, isem):
                pltpu.make_async_copy(
                    ei_hbm.at[:, pl.ds(astart + k * _B, _B)], eb, isem).wait()

            def mask(k, eb):
                for j in range(_B // _LANES):
                    pos = lane + (k * _B + j * _LANES)
                    ok = (pos >= shift) & (pos < total)
                    dv = eb[1, pl.ds(j * _LANES, _LANES)]
                    eb[1, pl.ds(j * _LANES, _LANES)] = jnp.where(ok, dv, trash)

            def step(k, eb_k, eb_n, rows, gsem, ssem, isem_k):
                pltpu.make_async_copy(h_hbm.at[eb_k.at[0]], rows, gsem).wait()
                pltpu.async_copy(rows, agg_sp.at[eb_k.at[1]], ssem, add=True)
                if with_deg:
                    pltpu.async_copy(
                        ones_v, deg_sp.at[eb_k.at[1]], ssem, add=True)
                pltpu.make_async_copy(rows, agg_sp.at[eb_k.at[1]], ssem).wait()
                if with_deg:
                    pltpu.make_async_copy(
                        ones_v, deg_sp.at[eb_k.at[1]], ssem).wait()
                pltpu.async_copy(h_hbm.at[eb_n.at[0]], rows, gsem)
                fire_idx(k + 4, eb_k, isem_k)
                mask(k + 2, eb_n)

            fire_idx(0, eb0, i0)
            fire_idx(1, eb1, i1)
            wait_idx(0, eb0, i0)
            pltpu.async_copy(h_hbm.at[eb0.at[0]], rows0, g0)
            fire_idx(2, eb2, i2)
            mask(0, eb0)
            wait_idx(1, eb1, i1)
            pltpu.async_copy(h_hbm.at[eb1.at[0]], rows1, g1)
            fire_idx(3, eb3, i3)
            mask(1, eb1)

            def quad(i, carry):
                k = 4 * i
                wait_idx(k + 2, eb2, i2)
                step(k, eb0, eb2, rows0, g0, s0, i0)
                wait_idx(k + 3, eb3, i3)
                step(k + 1, eb1, eb3, rows1, g1, s1, i1)
                wait_idx(k + 4, eb0, i0)
                step(k + 2, eb2, eb0, rows0, g0, s0, i2)
                wait_idx(k + 5, eb1, i1)
                step(k + 3, eb3, eb1, rows1, g1, s1, i3)
                return carry

            lax.fori_loop(0, nq, quad, 0)

            pltpu.make_async_copy(h_hbm.at[eb0.at[0]], rows0, g0).wait()
            pltpu.async_copy(rows0, agg_sp.at[eb0.at[1]], s0, add=True)
            pltpu.make_async_copy(h_hbm.at[eb1.at[0]], rows1, g1).wait()
            pltpu.async_copy(rows1, agg_sp.at[eb1.at[1]], s1, add=True)
            pltpu.make_async_copy(rows0, agg_sp.at[eb0.at[1]], s0).wait()
            pltpu.make_async_copy(rows1, agg_sp.at[eb1.at[1]], s1).wait()
            if with_deg:
                pltpu.async_copy(ones_v, deg_sp.at[eb0.at[1]], s0, add=True)
                pltpu.async_copy(ones_v, deg_sp.at[eb1.at[1]], s1, add=True)
                pltpu.make_async_copy(ones_v, deg_sp.at[eb0.at[1]], s0).wait()
                pltpu.make_async_copy(ones_v, deg_sp.at[eb1.at[1]], s1).wait()
            wait_idx(4 * nq + 2, eb2, i2)
            wait_idx(4 * nq + 3, eb3, i3)
            plsc.subcore_barrier()

            pltpu.sync_copy(agg_sp.at[pl.ds(row0, rpt)],
                            agg_hbm.at[r, pl.ds(row0, rpt)])
            if with_deg:
                pltpu.sync_copy(deg_sp.at[pl.ds(row0, rpt)],
                                deg_hbm.at[r, pl.ds(row0, rpt)])
            plsc.subcore_barrier()

    return sc_agg


def _make_tc_dense(n, n_pad, d_in, d_out, r_rel, nb_basis, blk, act):
    grid = n // blk

    def body(h_ref, agg_ref, deg_ref, v_ref, c_ref, l_ref, b_ref, o_ref):
        acc = jnp.dot(h_ref[...].astype(jnp.bfloat16),
                      l_ref[...].astype(jnp.bfloat16),
                      preferred_element_type=jnp.float32)
        acc = acc + b_ref[...]
        for r in range(r_rel):
            w = c_ref[r, 0] * v_ref[0]
            for bi in range(1, nb_basis):
                w = w + c_ref[r, bi] * v_ref[bi]
            deg = deg_ref[r][:, 0:1]
            a = agg_ref[r] / jnp.maximum(deg, 1.0)
            acc = acc + jnp.dot(a.astype(jnp.bfloat16),
                                w.astype(jnp.bfloat16),
                                preferred_element_type=jnp.float32)
        if act:
            acc = jnp.maximum(acc, 0.0)
        o_ref[...] = acc

    return pl.pallas_call(
        body,
        grid=(grid,),
        in_specs=[
            pl.BlockSpec((blk, d_in), lambda i: (i, 0)),
            pl.BlockSpec((r_rel, blk, d_in), lambda i: (0, i, 0)),
            pl.BlockSpec((r_rel, blk, _LANES), lambda i: (0, i, 0)),
            pl.BlockSpec((nb_basis, d_in, d_out), lambda i: (0, 0, 0)),
            pl.BlockSpec(memory_space=pltpu.SMEM),
            pl.BlockSpec((d_in, d_out), lambda i: (0, 0)),
            pl.BlockSpec((1, d_out), lambda i: (0, 0)),
        ],
        out_specs=pl.BlockSpec((blk, d_out), lambda i: (i, 0)),
        out_shape=jax.ShapeDtypeStruct((n, d_out), jnp.float32),
    )


def kernel(x, edge_index, edge_type, V1, c1, L1, b1, V2, c2, L2, b2,
           V3, c3, L3, b3):
    n, d_in = x.shape
    e = edge_index.shape[1]
    r_rel, nb_basis = c1.shape
    d_out = L3.shape[1]
    rpt = ((n + 8 + _NS * 8 - 1) // (_NS * 8)) * 8
    n_pad = rpt * _NS
    e_pad = e + 4096

    pad = jnp.stack([jnp.zeros((e_pad - e,), edge_index.dtype),
                     jnp.full((e_pad - e,), n, edge_index.dtype)])
    ei_pad = jnp.concatenate([edge_index, pad], axis=1)
    offs = jnp.sum(edge_type[None, :] < jnp.arange(r_rel + 1)[:, None],
                   axis=1, dtype=jnp.int32)
    offs16 = jnp.zeros((32,), jnp.int32).at[: r_rel + 1].set(offs)

    sc_agg_d = _make_sc_agg(n, e, d_in, r_rel, n_pad, rpt, True)
    sc_agg = _make_sc_agg(n, e, d_in, r_rel, n_pad, rpt, False)
    blk = 1000 if n % 1000 == 0 else 8

    layers = [(V1, c1, L1, b1, True),
              (V2, c2, L2, b2, True),
              (V3, c3, L3, b3, False)]
    h, deg = x, None
    for li, (V, c, L, b, act) in enumerate(layers):
        if li == 0:
            agg, deg = sc_agg_d(h, ei_pad, offs16)
        else:
            (agg,) = sc_agg(h, ei_pad, offs16)
        tc = _make_tc_dense(n, n_pad, d_in, d_out, r_rel, nb_basis, blk, act)
        h = tc(h, agg, deg, V, c, L, b.reshape(1, -1))
    return h


# speedup vs baseline: 1.5046x; 1.0193x over previous
import functools

import jax
import jax.numpy as jnp
from jax import lax
from jax.experimental import pallas as pl
from jax.experimental.pallas import tpu as pltpu
from jax.experimental.pallas import tpu_sc as plsc

_B = 128
_LANES = 16
_NS = 16
_NC = 2


def _make_sc_agg(n, e, d, r_rel, n_pad, rpt, with_deg):
    trash = n
    n_rounds = r_rel // _NC

    mesh = plsc.VectorSubcoreMesh(core_axis_name="c", subcore_axis_name="s")

    out_type = [jax.ShapeDtypeStruct((r_rel, n_pad, d), jnp.float32)]
    scratch = [
        pltpu.VMEM((32,), jnp.int32),
        pltpu.VMEM((2, _B), jnp.int32),
        pltpu.VMEM((2, _B), jnp.int32),
        pltpu.VMEM((2, _B), jnp.int32),
        pltpu.VMEM((2, _B), jnp.int32),
        pltpu.VMEM((_B, d), jnp.float32),
        pltpu.VMEM((_B, d), jnp.float32),
        pltpu.VMEM_SHARED((n_pad, d), jnp.float32),
        pltpu.SemaphoreType.DMA,
        pltpu.SemaphoreType.DMA,
        pltpu.SemaphoreType.DMA,
        pltpu.SemaphoreType.DMA,
        pltpu.SemaphoreType.DMA,
        pltpu.SemaphoreType.DMA,
        pltpu.SemaphoreType.DMA,
        pltpu.SemaphoreType.DMA,
    ]
    if with_deg:
        out_type.append(
            jax.ShapeDtypeStruct((r_rel, n_pad, _LANES), jnp.float32))
        scratch.append(pltpu.VMEM((_B, _LANES), jnp.float32))
        scratch.append(pltpu.VMEM_SHARED((n_pad, _LANES), jnp.float32))

    @functools.partial(
        pl.kernel,
        out_type=tuple(out_type),
        mesh=mesh,
        compiler_params=pltpu.CompilerParams(use_tc_tiling_on_sc=False),
        scratch_types=scratch,
    )
    def sc_agg(h_hbm, ei_hbm, offs_hbm, agg_hbm, *rest):
        if with_deg:
            (deg_hbm, offs_v, eb0, eb1, eb2, eb3, rows0, rows1, agg_sp,
             g0, g1, s0, s1, i0, i1, i2, i3, ones_v, deg_sp) = rest
        else:
            (offs_v, eb0, eb1, eb2, eb3, rows0, rows1, agg_sp,
             g0, g1, s0, s1, i0, i1, i2, i3) = rest
        c = lax.axis_index("c")
        s = lax.axis_index("s")
        lane = lax.iota(jnp.int32, _LANES)

        pltpu.sync_copy(offs_hbm, offs_v)
        zvec = jnp.zeros((_LANES,), jnp.float32)

        if with_deg:
            onev = jnp.ones((_LANES,), jnp.float32)

            def onerow(i, carry):
                ones_v[i, pl.ds(0, _LANES)] = onev
                return carry

            lax.fori_loop(0, _B, onerow, 0)

        for rnd in range(n_rounds):
            r = rnd * _NC + c

            def zrow(i, carry):
                for j in range(d // _LANES):
                    rows0[i, pl.ds(j * _LANES, _LANES)] = zvec
                return carry

            lax.fori_loop(0, _B, zrow, 0)
            row0 = s * rpt
            nfull = rpt // _B
            rem = rpt - nfull * _B
            for kz in range(nfull):
                pltpu.sync_copy(rows0, agg_sp.at[pl.ds(row0 + kz * _B, _B)])
                if with_deg:
                    pltpu.sync_copy(rows0.at[:, pl.ds(0, _LANES)],
                                    deg_sp.at[pl.ds(row0 + kz * _B, _B)])
            if rem:
                pltpu.sync_copy(rows0.at[pl.ds(0, rem)],
                                agg_sp.at[pl.ds(row0 + nfull * _B, rem)])
                if with_deg:
                    pltpu.sync_copy(rows0.at[pl.ds(0, rem), pl.ds(0, _LANES)],
                                    deg_sp.at[pl.ds(row0 + nfull * _B, rem)])
            plsc.subcore_barrier()

            ov = offs_v[pl.ds(r, _LANES)]
            off = ov[0]
            cnt = ov[1] - off
            per_tile = (cnt + _NS - 1) // _NS
            my_start = off + s * per_tile
            my_cnt = jnp.maximum(jnp.minimum(per_tile, cnt - s * per_tile), 0)
            astart = (my_start // 8) * 8
            shift = my_start - astart
            total = shift + my_cnt
            nb = (total + _B - 1) // _B
            nq = jnp.maximum((nb + 3) // 4, 1)

            def fire_idx(k, eb, isem):
                pltpu.async_copy(
                    ei_hbm.at[:, pl.ds(astart + k * _B, _B)], eb, isem)

            def wait_idx(k, eb, isem):
                pltpu.make_async_copy(
                    ei_hbm.at[:, pl.ds(astart + k * _B, _B)], eb, isem).wait()

            def mask(k, eb):
                for j in range(_B // _LANES):
                    pos = lane + (k * _B + j * _LANES)
                    ok = (pos >= shift) & (pos < total)
                    dv = eb[1, pl.ds(j * _LANES, _LANES)]
                    eb[1, pl.ds(j * _LANES, _LANES)] = jnp.where(ok, dv, trash)

            def step(k, eb_k, eb_n, rows, gsem, ssem, isem_k):
                pltpu.make_async_copy(h_hbm.at[eb_k.at[0]], rows, gsem).wait()
                pltpu.async_copy(rows, agg_sp.at[eb_k.at[1]], ssem, add=True)
                if with_deg:
                    pltpu.async_copy(
                        ones_v, deg_sp.at[eb_k.at[1]], ssem, add=True)
                pltpu.make_async_copy(
                    rows, agg_sp.at[eb_k.at[1]], ssem).wait()
                if with_deg:
                    pltpu.make_async_copy(
                        ones_v, deg_sp.at[eb_k.at[1]], ssem).wait()
                pltpu.async_copy(h_hbm.at[eb_n.at[0]], rows, gsem)
                fire_idx(k + 4, eb_k, isem_k)
                mask(k + 2, eb_n)

            fire_idx(0, eb0, i0)
            fire_idx(1, eb1, i1)
            wait_idx(0, eb0, i0)
            pltpu.async_copy(h_hbm.at[eb0.at[0]], rows0, g0)
            fire_idx(2, eb2, i2)
            mask(0, eb0)
            wait_idx(1, eb1, i1)
            pltpu.async_copy(h_hbm.at[eb1.at[0]], rows1, g1)
            fire_idx(3, eb3, i3)
            mask(1, eb1)

            def quad(i, carry):
                k = 4 * i
                wait_idx(k + 2, eb2, i2)
                step(k, eb0, eb2, rows0, g0, s0, i0)
                wait_idx(k + 3, eb3, i3)
                step(k + 1, eb1, eb3, rows1, g1, s1, i1)
                wait_idx(k + 4, eb0, i0)
                step(k + 2, eb2, eb0, rows0, g0, s0, i2)
                wait_idx(k + 5, eb1, i1)
                step(k + 3, eb3, eb1, rows1, g1, s1, i3)
                return carry

            lax.fori_loop(0, nq, quad, 0)

            pltpu.make_async_copy(h_hbm.at[eb0.at[0]], rows0, g0).wait()
            pltpu.make_async_copy(h_hbm.at[eb1.at[0]], rows1, g1).wait()
            wait_idx(4 * nq + 2, eb2, i2)
            wait_idx(4 * nq + 3, eb3, i3)
            plsc.subcore_barrier()

            pltpu.sync_copy(agg_sp.at[pl.ds(row0, rpt)],
                            agg_hbm.at[r, pl.ds(row0, rpt)])
            if with_deg:
                pltpu.sync_copy(deg_sp.at[pl.ds(row0, rpt)],
                                deg_hbm.at[r, pl.ds(row0, rpt)])
            plsc.subcore_barrier()

    return sc_agg


def _make_tc_dense(n, n_pad, d_in, d_out, r_rel, nb_basis, blk, act):
    grid = n // blk

    def body(h_ref, agg_ref, deg_ref, v_ref, c_ref, l_ref, b_ref, o_ref):
        acc = jnp.dot(h_ref[...].astype(jnp.bfloat16),
                      l_ref[...].astype(jnp.bfloat16),
                      preferred_element_type=jnp.float32)
        acc = acc + b_ref[...]
        for r in range(r_rel):
            w = c_ref[r, 0] * v_ref[0]
            for bi in range(1, nb_basis):
                w = w + c_ref[r, bi] * v_ref[bi]
            deg = deg_ref[r][:, 0:1]
            a = agg_ref[r] / jnp.maximum(deg, 1.0)
            acc = acc + jnp.dot(a.astype(jnp.bfloat16),
                                w.astype(jnp.bfloat16),
                                preferred_element_type=jnp.float32)
        if act:
            acc = jnp.maximum(acc, 0.0)
        o_ref[...] = acc

    return pl.pallas_call(
        body,
        grid=(grid,),
        in_specs=[
            pl.BlockSpec((blk, d_in), lambda i: (i, 0)),
            pl.BlockSpec((r_rel, blk, d_in), lambda i: (0, i, 0)),
            pl.BlockSpec((r_rel, blk, _LANES), lambda i: (0, i, 0)),
            pl.BlockSpec((nb_basis, d_in, d_out), lambda i: (0, 0, 0)),
            pl.BlockSpec(memory_space=pltpu.SMEM),
            pl.BlockSpec((d_in, d_out), lambda i: (0, 0)),
            pl.BlockSpec((1, d_out), lambda i: (0, 0)),
        ],
        out_specs=pl.BlockSpec((blk, d_out), lambda i: (i, 0)),
        out_shape=jax.ShapeDtypeStruct((n, d_out), jnp.float32),
    )


def kernel(x, edge_index, edge_type, V1, c1, L1, b1, V2, c2, L2, b2,
           V3, c3, L3, b3):
    n, d_in = x.shape
    e = edge_index.shape[1]
    r_rel, nb_basis = c1.shape
    d_out = L3.shape[1]
    rpt = ((n + 8 + _NS * 8 - 1) // (_NS * 8)) * 8
    n_pad = rpt * _NS
    e_pad = e + 4096

    pad = jnp.stack([jnp.zeros((e_pad - e,), edge_index.dtype),
                     jnp.full((e_pad - e,), n, edge_index.dtype)])
    ei_pad = jnp.concatenate([edge_index, pad], axis=1)
    offs = jnp.sum(edge_type[None, :] < jnp.arange(r_rel + 1)[:, None],
                   axis=1, dtype=jnp.int32)
    offs16 = jnp.zeros((32,), jnp.int32).at[: r_rel + 1].set(offs)

    sc_agg_d = _make_sc_agg(n, e, d_in, r_rel, n_pad, rpt, True)
    sc_agg = _make_sc_agg(n, e, d_in, r_rel, n_pad, rpt, False)
    blk = 1000 if n % 1000 == 0 else 8

    layers = [(V1, c1, L1, b1, True),
              (V2, c2, L2, b2, True),
              (V3, c3, L3, b3, False)]
    h, deg = x, None
    for li, (V, c, L, b, act) in enumerate(layers):
        if li == 0:
            agg, deg = sc_agg_d(h, ei_pad, offs16)
        else:
            (agg,) = sc_agg(h, ei_pad, offs16)
        tc = _make_tc_dense(n, n_pad, d_in, d_out, r_rel, nb_basis, blk, act)
        h = tc(h, agg, deg, V, c, L, b.reshape(1, -1))
    return h
